# baseline (speedup 1.0000x reference)
import jax
import jax.numpy as jnp
from jax.experimental import pallas as pl
from jax.experimental.pallas import tpu as pltpu

_BF = jnp.bfloat16
_F32 = jnp.float32


def _build_im1(srcs):
    cols = []
    for src, sh in srcs:
        col = jax.lax.slice(src, (sh, 0), (512, 8))
        if sh:
            col = jnp.pad(col, ((0, sh), (0, 0)))
        cols.append(col)
    xts = jnp.concatenate(cols, axis=1)
    return jnp.concatenate(
        [jax.lax.slice(xts, (16 * kh, 0), (16 * kh + 448, 40))
         for kh in range(5)],
        axis=1)


def _one_image(xr, w1, b1, w2, b2):
    xe = xr[:, 0:8]
    xo = xr[:, 8:16]
    ime = _build_im1([(xe, 0), (xo, 0), (xe, 1), (xo, 1), (xe, 2)])
    imo = _build_im1([(xo, 0), (xe, 1), (xo, 1), (xe, 2), (xo, 2)])
    pe = jnp.maximum(
        jnp.dot(ime, w1, preferred_element_type=_F32) + b1, 0.0).astype(_BF)
    po = jnp.maximum(
        jnp.dot(imo, w1, preferred_element_type=_F32) + b1, 0.0).astype(_BF)
    p1 = jnp.maximum(
        jnp.maximum(pe[0:448, :], po[0:448, :]),
        jnp.maximum(jnp.pad(pe[16:448, :], ((0, 16), (0, 0))),
                    jnp.pad(po[16:448, :], ((0, 16), (0, 0)))))
    p1c = jax.lax.slice(p1.reshape(14, 32, 64), (0, 0, 0),
                        (14, 16, 64)).reshape(224, 64)
    im2 = jnp.concatenate(
        [jax.lax.slice(p1c, (16 * kh + kw, 0), (16 * kh + kw + 154, 64))
         for kh in range(5) for kw in range(5)],
        axis=1)
    c2 = jnp.dot(im2, w2, preferred_element_type=_F32)
    c3 = c2.reshape(77, 2, 64)
    se = c3[:, 0, :]
    so = c3[:, 1, :]
    q = jnp.maximum(
        jnp.maximum(se[0:69, :], so[0:69, :]),
        jnp.maximum(se[8:77, :], so[8:77, :]))
    rows = [jax.lax.slice(q, (16 * hp, 0), (16 * hp + 5, 64))
            for hp in range(5)]
    feats = jnp.concatenate(rows, axis=0)
    return jnp.maximum(feats + b2, 0.0).astype(_BF)


def _conv_tower_kernel(x_ref, w1_ref, b1_ref, w2_ref, b2_ref, f_ref):
    bn = x_ref.shape[0]
    w1 = w1_ref[...]
    b1 = b1_ref[...]
    w2 = w2_ref[...]
    b2 = b2_ref[...]

    def body(i, carry):
        for g in range(4):
            f_ref[4 * i + g] = _one_image(x_ref[4 * i + g], w1, b1, w2, b2)
        return carry

    jax.lax.fori_loop(0, bn // 4, body, 0)


def _conv_tower(x_bf, w1, b1, w2, b2):
    B = x_bf.shape[0]
    bn = 1
    for cand in (16, 8, 4, 2):
        if B % cand == 0:
            bn = cand
            break
    half = B // bn // 2
    return pl.pallas_call(
        _conv_tower_kernel,
        out_shape=jax.ShapeDtypeStruct((B, 25, 64), _BF),
        grid=(2, half),
        in_specs=[
            pl.BlockSpec((bn, 512, 16), lambda c, b: (c * half + b, 0, 0)),
            pl.BlockSpec((200, 64), lambda c, b: (0, 0)),
            pl.BlockSpec((1, 64), lambda c, b: (0, 0)),
            pl.BlockSpec((1600, 64), lambda c, b: (0, 0)),
            pl.BlockSpec((1, 64), lambda c, b: (0, 0)),
        ],
        out_specs=pl.BlockSpec((bn, 25, 64), lambda c, b: (c * half + b, 0, 0)),
        compiler_params=pltpu.CompilerParams(
            dimension_semantics=("parallel", "arbitrary"),
            allow_input_fusion=[True, False, False, False, False]),
    )(x_bf, w1, b1, w2, b2)


def _fc_stack_kernel(x_ref, w1_ref, b1_ref, w2_ref, b2_ref, w3_ref, b3_ref,
                     o_ref):
    h = jnp.dot(x_ref[...], w1_ref[...], preferred_element_type=_F32)
    h = jnp.maximum(h + b1_ref[...], 0.0).astype(_BF)
    h = jnp.dot(h, w2_ref[...], preferred_element_type=_F32)
    h = jnp.maximum(h + b2_ref[...], 0.0).astype(_BF)
    o_ref[...] = jnp.dot(h, w3_ref[...], preferred_element_type=_F32) \
        + b3_ref[...]


def _fc_stack(feats2d, w1, b1, w2, b2, w3, b3):
    B = feats2d.shape[0]
    tm = B
    for cand in (256, 128, 64):
        if B % cand == 0:
            tm = cand
            break
    halffc = B // tm // 2
    return pl.pallas_call(
        _fc_stack_kernel,
        out_shape=jax.ShapeDtypeStruct((B, 128), _F32),
        grid=(2, halffc),
        in_specs=[
            pl.BlockSpec((tm, 1600), lambda c, i: (c * halffc + i, 0)),
            pl.BlockSpec((1600, 384), lambda c, i: (0, 0)),
            pl.BlockSpec((1, 384), lambda c, i: (0, 0)),
            pl.BlockSpec((384, 192), lambda c, i: (0, 0)),
            pl.BlockSpec((1, 192), lambda c, i: (0, 0)),
            pl.BlockSpec((192, 128), lambda c, i: (0, 0)),
            pl.BlockSpec((1, 128), lambda c, i: (0, 0)),
        ],
        out_specs=pl.BlockSpec((tm, 128), lambda c, i: (c * halffc + i, 0)),
        compiler_params=pltpu.CompilerParams(
            dimension_semantics=("parallel", "arbitrary")),
    )(feats2d, w1, b1, w2, b2, w3, b3)


@jax.jit
def _forward(x_nchw, w1k, b1r, w2k, b2r, fc1a_w, fc1_b, fc2a_w, fc2_b,
             fc3a_w, fc3_b):
    B = x_nchw.shape[0]
    x = jnp.transpose(x_nchw.astype(_BF), (0, 2, 3, 1)).reshape(B, 1024, 3)
    x = jnp.pad(x, ((0, 0), (0, 0), (0, 5)))
    x = x.reshape(B, 512, 16)
    w1 = w1k.reshape(200, 64).astype(_BF)
    w2 = w2k.reshape(1600, 64).astype(_BF)
    feats = _conv_tower(x, w1, b1r, w2, b2r)
    logits = _fc_stack(feats.reshape(B, 1600),
                       fc1a_w.astype(_BF), fc1_b,
                       fc2a_w.astype(_BF), fc2_b,
                       fc3a_w.astype(_BF), fc3_b)
    return logits[:, :10]


def kernel(x_nchw, w1k, b1r, w2k, b2r, fc1a_w, fc1_b, fc2a_w, fc2_b,
           fc3a_w, fc3_b):
    return _forward(x_nchw, w1k, b1r, w2k, b2r, fc1a_w, fc1_b, fc2a_w,
                    fc2_b, fc3a_w, fc3_b)


# speedup vs baseline: 1.0031x; 1.0031x over previous
import jax
import jax.numpy as jnp
from jax.experimental import pallas as pl
from jax.experimental.pallas import tpu as pltpu

_BF = jnp.bfloat16
_F32 = jnp.float32


def _build_im1(srcs):
    cols = []
    for src, sh in srcs:
        col = jax.lax.slice(src, (sh, 0), (512, 8))
        if sh:
            col = jnp.pad(col, ((0, sh), (0, 0)))
        cols.append(col)
    xts = jnp.concatenate(cols, axis=1)
    return jnp.concatenate(
        [jax.lax.slice(xts, (16 * kh, 0), (16 * kh + 448, 40))
         for kh in range(5)],
        axis=1)


def _one_image(xr, w1, b1, w2, b2):
    xe = xr[:, 0:8]
    xo = xr[:, 8:16]
    ime = _build_im1([(xe, 0), (xo, 0), (xe, 1), (xo, 1), (xe, 2)])
    imo = _build_im1([(xo, 0), (xe, 1), (xo, 1), (xe, 2), (xo, 2)])
    pe = jnp.maximum(
        jnp.dot(ime, w1, preferred_element_type=_F32) + b1, 0.0).astype(_BF)
    po = jnp.maximum(
        jnp.dot(imo, w1, preferred_element_type=_F32) + b1, 0.0).astype(_BF)
    p1 = jnp.pad(jnp.maximum(
        jnp.maximum(pe[0:432, :], po[0:432, :]),
        jnp.maximum(pe[16:448, :], po[16:448, :])), ((0, 16), (0, 0)))
    p1c = jax.lax.slice(p1.reshape(14, 32, 64), (0, 0, 0),
                        (14, 16, 64)).reshape(224, 64)
    im2 = jnp.concatenate(
        [jax.lax.slice(p1c, (16 * kh + kw, 0), (16 * kh + kw + 154, 64))
         for kh in range(5) for kw in range(5)],
        axis=1)
    c2 = jnp.dot(im2, w2, preferred_element_type=_F32)
    c3 = c2.reshape(77, 2, 64)
    se = c3[:, 0, :]
    so = c3[:, 1, :]
    q = jnp.maximum(
        jnp.maximum(se[0:69, :], so[0:69, :]),
        jnp.maximum(se[8:77, :], so[8:77, :]))
    rows = [jax.lax.slice(q, (16 * hp, 0), (16 * hp + 5, 64))
            for hp in range(5)]
    feats = jnp.concatenate(rows, axis=0)
    return jnp.maximum(feats + b2, 0.0).astype(_BF)


def _conv_tower_kernel(x_ref, w1_ref, b1_ref, w2_ref, b2_ref, f_ref):
    bn = x_ref.shape[0]
    w1 = w1_ref[...]
    b1 = b1_ref[...]
    w2 = w2_ref[...]
    b2 = b2_ref[...]

    def body(i, carry):
        for g in range(4):
            f_ref[4 * i + g] = _one_image(x_ref[4 * i + g], w1, b1, w2, b2)
        return carry

    jax.lax.fori_loop(0, bn // 4, body, 0)


def _conv_tower(x_bf, w1, b1, w2, b2):
    B = x_bf.shape[0]
    bn = 1
    for cand in (64, 32, 16, 8, 4, 2):
        if B % cand == 0:
            bn = cand
            break
    half = B // bn // 2
    return pl.pallas_call(
        _conv_tower_kernel,
        out_shape=jax.ShapeDtypeStruct((B, 25, 64), _BF),
        grid=(2, half),
        in_specs=[
            pl.BlockSpec((bn, 512, 16), lambda c, b: (c * half + b, 0, 0)),
            pl.BlockSpec((200, 64), lambda c, b: (0, 0)),
            pl.BlockSpec((1, 64), lambda c, b: (0, 0)),
            pl.BlockSpec((1600, 64), lambda c, b: (0, 0)),
            pl.BlockSpec((1, 64), lambda c, b: (0, 0)),
        ],
        out_specs=pl.BlockSpec((bn, 25, 64), lambda c, b: (c * half + b, 0, 0)),
        compiler_params=pltpu.CompilerParams(
            dimension_semantics=("parallel", "arbitrary")),
    )(x_bf, w1, b1, w2, b2)


def _fc_stack_kernel(x_ref, w1_ref, b1_ref, w2_ref, b2_ref, w3_ref, b3_ref,
                     o_ref):
    h = jnp.dot(x_ref[...], w1_ref[...], preferred_element_type=_F32)
    h = jnp.maximum(h + b1_ref[...], 0.0).astype(_BF)
    h = jnp.dot(h, w2_ref[...], preferred_element_type=_F32)
    h = jnp.maximum(h + b2_ref[...], 0.0).astype(_BF)
    o_ref[...] = jnp.dot(h, w3_ref[...], preferred_element_type=_F32) \
        + b3_ref[...]


def _fc_stack(feats2d, w1, b1, w2, b2, w3, b3):
    B = feats2d.shape[0]
    tm = B
    for cand in (256, 128, 64):
        if B % cand == 0:
            tm = cand
            break
    halffc = B // tm // 2
    return pl.pallas_call(
        _fc_stack_kernel,
        out_shape=jax.ShapeDtypeStruct((B, 128), _F32),
        grid=(2, halffc),
        in_specs=[
            pl.BlockSpec((tm, 1600), lambda c, i: (c * halffc + i, 0)),
            pl.BlockSpec((1600, 384), lambda c, i: (0, 0)),
            pl.BlockSpec((1, 384), lambda c, i: (0, 0)),
            pl.BlockSpec((384, 192), lambda c, i: (0, 0)),
            pl.BlockSpec((1, 192), lambda c, i: (0, 0)),
            pl.BlockSpec((192, 128), lambda c, i: (0, 0)),
            pl.BlockSpec((1, 128), lambda c, i: (0, 0)),
        ],
        out_specs=pl.BlockSpec((tm, 128), lambda c, i: (c * halffc + i, 0)),
        compiler_params=pltpu.CompilerParams(
            dimension_semantics=("parallel", "arbitrary")),
    )(feats2d, w1, b1, w2, b2, w3, b3)


@jax.jit
def _forward(x_nchw, w1k, b1r, w2k, b2r, fc1a_w, fc1_b, fc2a_w, fc2_b,
             fc3a_w, fc3_b):
    B = x_nchw.shape[0]
    x = jnp.transpose(x_nchw.astype(_BF), (0, 2, 3, 1)).reshape(B, 1024, 3)
    x = jnp.pad(x, ((0, 0), (0, 0), (0, 5)))
    x = x.reshape(B, 512, 16)
    w1 = w1k.reshape(200, 64).astype(_BF)
    w2 = w2k.reshape(1600, 64).astype(_BF)
    feats = _conv_tower(x, w1, b1r, w2, b2r)
    logits = _fc_stack(feats.reshape(B, 1600),
                       fc1a_w.astype(_BF), fc1_b,
                       fc2a_w.astype(_BF), fc2_b,
                       fc3a_w.astype(_BF), fc3_b)
    return logits[:, :10]


def kernel(x_nchw, w1k, b1r, w2k, b2r, fc1a_w, fc1_b, fc2a_w, fc2_b,
           fc3a_w, fc3_b):
    return _forward(x_nchw, w1k, b1r, w2k, b2r, fc1a_w, fc1_b, fc2a_w,
                    fc2_b, fc3a_w, fc3_b)
